# baseline (speedup 1.0000x reference)
import functools

import jax
import jax.numpy as jnp
from jax import lax
from jax.experimental import pallas as pl
from jax.experimental.pallas import tpu as pltpu
from jax.experimental.pallas import tpu_sc as plsc

NUM_CORES = 2
NUM_SUBCORES = 16
LANES = 16
DPAD = 128
RA = 104
RB = 96


def _half_kernel(inputs_flat, tab128, pos_table, B, S, D):
    NW = NUM_CORES * NUM_SUBCORES
    seqs_per_w = B // NW
    idx_per_w = seqs_per_w * S

    mesh = plsc.VectorSubcoreMesh(core_axis_name="c", subcore_axis_name="s")

    @functools.partial(
        pl.kernel,
        mesh=mesh,
        out_type=jax.ShapeDtypeStruct((B, S, D), jnp.float32),
        scratch_types=[
            pltpu.VMEM((idx_per_w,), jnp.int32),
            pltpu.VMEM((2, RA, DPAD), jnp.float32),
            pltpu.VMEM((2, RA, D), jnp.float32),
            pltpu.VMEM((S, D), jnp.float32),
        ]
        + [pltpu.SemaphoreType.DMA] * 4,
    )
    def emb_kernel(inp_hbm, tab_hbm, pos_hbm, out_hbm, idx_v, rows_v, st_v, pos_v, *sems):
        gsem = sems[:2]
        ssem = sems[2:]
        wid = lax.axis_index("s") * NUM_CORES + lax.axis_index("c")
        base_idx = wid * idx_per_w
        base_seq = wid * seqs_per_w

        pltpu.sync_copy(pos_hbm, pos_v)
        pltpu.sync_copy(
            inp_hbm.at[pl.ds(pl.multiple_of(base_idx, 128), idx_per_w)], idx_v
        )

        def fire(s, part):
            r0, n = (0, RA) if part == 0 else (RA, RB)
            off = pl.multiple_of(s * S + r0, 8)
            pltpu.async_copy(
                tab_hbm.at[idx_v.at[pl.ds(off, n)]],
                rows_v.at[part, pl.ds(0, n)],
                gsem[part],
            )

        def wait_scatter(part):
            n = RA if part == 0 else RB
            pltpu.make_async_copy(
                st_v.at[part, pl.ds(0, n)],
                out_hbm.at[0, pl.ds(0, n)],
                ssem[part],
            ).wait()

        def process(s, part):
            r0, n = (0, RA) if part == 0 else (RA, RB)
            pltpu.make_async_copy(
                tab_hbm.at[pl.ds(0, n)], rows_v.at[part, pl.ds(0, n)], gsem[part]
            ).wait()

            def add_body(r, u):
                for ci in range(D // LANES):
                    sl = pl.ds(ci * LANES, LANES)
                    st_v[part, r, sl] = rows_v[part, r, sl] + pos_v[r0 + r, sl]
                return u

            lax.fori_loop(0, n, add_body, 0)
            pltpu.async_copy(
                st_v.at[part, pl.ds(0, n)],
                out_hbm.at[base_seq + s, pl.ds(r0, n)],
                ssem[part],
            )

        fire(0, 0)
        fire(0, 1)
        process(0, 0)
        fire(1, 0)
        process(0, 1)

        def super_body(s, carry):
            fire(s, 1)
            wait_scatter(0)
            process(s, 0)
            fire(s + 1, 0)
            wait_scatter(1)
            process(s, 1)
            return carry

        lax.fori_loop(1, seqs_per_w - 1, super_body, 0)

        s_last = seqs_per_w - 1
        fire(s_last, 1)
        wait_scatter(0)
        process(s_last, 0)
        wait_scatter(1)
        process(s_last, 1)
        wait_scatter(0)
        wait_scatter(1)

    return emb_kernel(inputs_flat, tab128, pos_table)


def kernel(inputs, token_table, pos_table):
    B, S = inputs.shape
    V, D = token_table.shape
    HB = B // 2

    tab128 = jnp.pad(token_table, ((0, 0), (0, DPAD - D)))
    flat = inputs.reshape(B * S)

    h0 = _half_kernel(flat[: HB * S], tab128, pos_table, HB, S, D)
    h1 = _half_kernel(flat[HB * S:], tab128, pos_table, HB, S, D)
    out = jnp.zeros((B, S, D), jnp.float32)
    out = lax.dynamic_update_slice(out, h0, (0, 0, 0))
    out = lax.dynamic_update_slice(out, h1, (HB, 0, 0))
    return out


# speedup vs baseline: 1.0726x; 1.0726x over previous
import functools

import jax
import jax.numpy as jnp
from jax import lax
from jax.experimental import pallas as pl
from jax.experimental.pallas import tpu as pltpu
from jax.experimental.pallas import tpu_sc as plsc

NUM_CORES = 2
NUM_SUBCORES = 16
LANES = 16
DPAD = 128
RA = 104
RB = 96


def kernel(inputs, token_table, pos_table):
    B, S = inputs.shape
    V, D = token_table.shape
    NW = NUM_CORES * NUM_SUBCORES
    seqs_per_w = B // NW
    idx_per_w = seqs_per_w * S

    tab128 = jnp.pad(token_table, ((0, 0), (0, DPAD - D)))
    pos128 = jnp.pad(pos_table, ((0, 0), (0, DPAD - D)))
    inputs_flat = inputs.reshape(B * S)

    mesh = plsc.VectorSubcoreMesh(core_axis_name="c", subcore_axis_name="s")

    @functools.partial(
        pl.kernel,
        mesh=mesh,
        out_type=jax.ShapeDtypeStruct((B, S, D), jnp.float32),
        compiler_params=pltpu.CompilerParams(use_tc_tiling_on_sc=False),
        scratch_types=[
            pltpu.VMEM((idx_per_w,), jnp.int32),
            pltpu.VMEM((4, RA, DPAD), jnp.float32),
            pltpu.VMEM((S, DPAD), jnp.float32),
        ]
        + [pltpu.SemaphoreType.DMA] * 8,
    )
    def emb_kernel(inp_hbm, tab_hbm, pos_hbm, out_hbm, idx_v, rows_v, pos_v, *sems):
        gsem = sems[:4]
        ssem = sems[4:]
        wid = lax.axis_index("s") * NUM_CORES + lax.axis_index("c")
        base_idx = wid * idx_per_w
        base_seq = wid * seqs_per_w

        pltpu.sync_copy(pos_hbm, pos_v)
        pltpu.sync_copy(
            inp_hbm.at[pl.ds(pl.multiple_of(base_idx, 128), idx_per_w)], idx_v
        )

        def part_of(b):
            return b % 2

        def fire(s, b):
            r0, n = (0, RA) if part_of(b) == 0 else (RA, RB)
            off = pl.multiple_of(s * S + r0, 8)
            pltpu.async_copy(
                tab_hbm.at[idx_v.at[pl.ds(off, n)]],
                rows_v.at[b, pl.ds(0, n)],
                gsem[b],
            )

        def wait_sc(b):
            n = RA if part_of(b) == 0 else RB
            pltpu.make_async_copy(
                rows_v.at[b, pl.ds(0, n), pl.ds(0, D)],
                out_hbm.at[0, pl.ds(0, n)],
                ssem[b],
            ).wait()

        def process(s, b):
            r0, n = (0, RA) if part_of(b) == 0 else (RA, RB)
            pltpu.make_async_copy(
                tab_hbm.at[pl.ds(0, n)], rows_v.at[b, pl.ds(0, n)], gsem[b]
            ).wait()

            def add_body(r, u):
                for ci in range(D // LANES):
                    sl = pl.ds(ci * LANES, LANES)
                    rows_v[b, r, sl] = rows_v[b, r, sl] + pos_v[r0 + r, sl]
                return u

            lax.fori_loop(0, n, add_body, 0)
            pltpu.async_copy(
                rows_v.at[b, pl.ds(0, n), pl.ds(0, D)],
                out_hbm.at[base_seq + s, pl.ds(r0, n)],
                ssem[b],
            )

        fire(0, 0)
        fire(0, 1)

        fire(1, 2)
        process(0, 0)
        fire(1, 3)
        process(0, 1)
        wait_sc(0)
        fire(2, 0)
        process(1, 2)
        wait_sc(1)
        fire(2, 1)
        process(1, 3)

        def super_body(k, carry):
            s0 = 2 * k
            wait_sc(2)
            fire(s0 + 1, 2)
            process(s0, 0)
            wait_sc(3)
            fire(s0 + 1, 3)
            process(s0, 1)
            wait_sc(0)
            fire(s0 + 2, 0)
            process(s0 + 1, 2)
            wait_sc(1)
            fire(s0 + 2, 1)
            process(s0 + 1, 3)
            return carry

        lax.fori_loop(1, seqs_per_w // 2 - 1, super_body, 0)

        s0 = seqs_per_w - 2
        wait_sc(2)
        fire(s0 + 1, 2)
        process(s0, 0)
        wait_sc(3)
        fire(s0 + 1, 3)
        process(s0, 1)
        process(s0 + 1, 2)
        process(s0 + 1, 3)
        for b in range(4):
            wait_sc(b)

    return emb_kernel(inputs_flat, tab128, pos128)


# speedup vs baseline: 1.1340x; 1.0573x over previous
import functools

import jax
import jax.numpy as jnp
from jax import lax
from jax.experimental import pallas as pl
from jax.experimental.pallas import tpu as pltpu
from jax.experimental.pallas import tpu_sc as plsc

NUM_CORES = 2
NUM_SUBCORES = 16
LANES = 16
DPAD = 128


def kernel(inputs, token_table, pos_table):
    B, S = inputs.shape
    V, D = token_table.shape
    NW = NUM_CORES * NUM_SUBCORES
    seqs_per_w = B // NW
    idx_per_w = seqs_per_w * S
    SP = S // 2

    tab128 = jnp.pad(token_table, ((0, 0), (0, DPAD - D)))
    pos_packed = pos_table.reshape(SP, 2 * D)
    inputs_flat = inputs.reshape(B * S)

    mesh = plsc.VectorSubcoreMesh(core_axis_name="c", subcore_axis_name="s")

    @functools.partial(
        pl.kernel,
        mesh=mesh,
        out_type=jax.ShapeDtypeStruct((B, SP, 2 * D), jnp.float32),
        scratch_types=[
            pltpu.VMEM((idx_per_w,), jnp.int32),
            pltpu.VMEM((2, S, DPAD), jnp.float32),
            pltpu.VMEM((2, SP, 2 * D), jnp.float32),
            pltpu.VMEM((SP, 2 * D), jnp.float32),
        ]
        + [pltpu.SemaphoreType.DMA] * 4,
    )
    def emb_kernel(inp_hbm, tab_hbm, pos_hbm, out_hbm, idx_v, rows_v, st_v, pos_v, *sems):
        gsem = sems[:2]
        ssem = sems[2:]
        wid = lax.axis_index("s") * NUM_CORES + lax.axis_index("c")
        base_idx = wid * idx_per_w
        base_seq = wid * seqs_per_w

        pltpu.sync_copy(pos_hbm, pos_v)
        pltpu.sync_copy(
            inp_hbm.at[pl.ds(pl.multiple_of(base_idx, 128), idx_per_w)], idx_v
        )

        def fire(s, b):
            off = pl.multiple_of(s * S, 8)
            pltpu.async_copy(
                tab_hbm.at[idx_v.at[pl.ds(off, 128)]],
                rows_v.at[b, pl.ds(0, 128)],
                gsem[b],
            )
            pltpu.async_copy(
                tab_hbm.at[idx_v.at[pl.ds(off + 128, S - 128)]],
                rows_v.at[b, pl.ds(128, S - 128)],
                gsem[b],
            )

        def wait_sc(b):
            pltpu.make_async_copy(st_v.at[b], out_hbm.at[0], ssem[b]).wait()

        def process(s, b):
            pltpu.make_async_copy(
                tab_hbm.at[pl.ds(0, S)], rows_v.at[b], gsem[b]
            ).wait()

            def add_body(r2, u):
                for half in range(2):
                    for ci in range(D // LANES):
                        dsl = pl.ds(half * D + ci * LANES, LANES)
                        ssl = pl.ds(ci * LANES, LANES)
                        st_v[b, r2, dsl] = rows_v[b, 2 * r2 + half, ssl] + pos_v[r2, dsl]
                return u

            lax.fori_loop(0, SP, add_body, 0)
            pltpu.async_copy(st_v.at[b], out_hbm.at[base_seq + s], ssem[b])

        fire(0, 0)
        fire(1, 1)
        process(0, 0)
        fire(2, 0)
        process(1, 1)

        def super_body(k, carry):
            s = 2 * k
            fire(s + 1, 1)
            wait_sc(0)
            process(s, 0)
            fire(s + 2, 0)
            wait_sc(1)
            process(s + 1, 1)
            return carry

        lax.fori_loop(1, seqs_per_w // 2 - 1, super_body, 0)

        s = seqs_per_w - 2
        fire(s + 1, 1)
        wait_sc(0)
        process(s, 0)
        wait_sc(1)
        process(s + 1, 1)
        wait_sc(0)
        wait_sc(1)

    out = emb_kernel(inputs_flat, tab128, pos_packed)
    return out.reshape(B, S, D)


# speedup vs baseline: 1.7783x; 1.5681x over previous
import functools

import jax
import jax.numpy as jnp
from jax import lax
from jax.experimental import pallas as pl
from jax.experimental.pallas import tpu as pltpu
from jax.experimental.pallas import tpu_sc as plsc

NUM_CORES = 2
NUM_SUBCORES = 16
LANES = 16
DPAD = 128


def kernel(inputs, token_table, pos_table):
    B, S = inputs.shape
    V, D = token_table.shape
    NW = NUM_CORES * NUM_SUBCORES
    seqs_per_w = B // NW
    idx_per_w = seqs_per_w * S
    SP = S // 2

    tab128 = jnp.pad(token_table, ((0, 0), (0, DPAD - D)))
    pos_packed = pos_table.reshape(SP, 2 * D)
    inputs_flat = inputs.reshape(B * S)

    mesh = plsc.VectorSubcoreMesh(core_axis_name="c", subcore_axis_name="s")

    @functools.partial(
        pl.kernel,
        mesh=mesh,
        out_type=jax.ShapeDtypeStruct((B, SP, 2 * D), jnp.float32),
        scratch_types=[
            pltpu.VMEM((idx_per_w,), jnp.int32),
            pltpu.VMEM((2, S, DPAD), jnp.float32),
            pltpu.VMEM((2, SP, 2 * D), jnp.float32),
            pltpu.VMEM((SP, 2 * D), jnp.float32),
        ]
        + [pltpu.SemaphoreType.DMA] * 4,
    )
    def emb_kernel(inp_hbm, tab_hbm, pos_hbm, out_hbm, idx_v, rows_v, st_v, pos_v, *sems):
        gsem = sems[:2]
        ssem = sems[2:]
        wid = lax.axis_index("s") * NUM_CORES + lax.axis_index("c")
        base_idx = wid * idx_per_w
        base_seq = wid * seqs_per_w

        pltpu.sync_copy(pos_hbm, pos_v)
        pltpu.sync_copy(
            inp_hbm.at[pl.ds(pl.multiple_of(base_idx, 128), idx_per_w)], idx_v
        )

        def fire(s, b):
            off = pl.multiple_of(s * S, 8)
            pltpu.async_copy(
                tab_hbm.at[idx_v.at[pl.ds(off, 128)]],
                rows_v.at[b, pl.ds(0, 128)],
                gsem[b],
            )
            pltpu.async_copy(
                tab_hbm.at[idx_v.at[pl.ds(off + 128, S - 128)]],
                rows_v.at[b, pl.ds(128, S - 128)],
                gsem[b],
            )

        def wait_sc(b):
            pltpu.make_async_copy(st_v.at[b], out_hbm.at[0], ssem[b]).wait()

        def process(s, b):
            pltpu.make_async_copy(
                tab_hbm.at[pl.ds(0, S)], rows_v.at[b], gsem[b]
            ).wait()

            @plsc.parallel_loop(0, SP, unroll=2)
            def add_body(r2):
                for half in range(2):
                    for ci in range(D // LANES):
                        dsl = pl.ds(half * D + ci * LANES, LANES)
                        ssl = pl.ds(ci * LANES, LANES)
                        st_v[b, r2, dsl] = rows_v[b, 2 * r2 + half, ssl] + pos_v[r2, dsl]
            pltpu.async_copy(st_v.at[b], out_hbm.at[base_seq + s], ssem[b])

        fire(0, 0)
        fire(1, 1)
        process(0, 0)
        fire(2, 0)
        process(1, 1)

        def super_body(k, carry):
            s = 2 * k
            fire(s + 1, 1)
            wait_sc(0)
            process(s, 0)
            fire(s + 2, 0)
            wait_sc(1)
            process(s + 1, 1)
            return carry

        lax.fori_loop(1, seqs_per_w // 2 - 1, super_body, 0)

        s = seqs_per_w - 2
        fire(s + 1, 1)
        wait_sc(0)
        process(s, 0)
        wait_sc(1)
        process(s + 1, 1)
        wait_sc(0)
        wait_sc(1)

    out = emb_kernel(inputs_flat, tab128, pos_packed)
    return out.reshape(B, S, D)
